# baseline (speedup 1.0000x reference)
import jax
import jax.numpy as jnp
from jax import lax
from jax.experimental import pallas as pl
from jax.experimental.pallas import tpu as pltpu
from jax.experimental.pallas import tpu_sc as plsc

_TOKENS = 49408
_D = 512
_S = 77
_B = 4096
_N = _B * _S

_info = plsc.get_sparse_core_info()
_NC, _NS, _L = _info.num_cores, _info.num_subcores, _info.num_lanes
_NW = _NC * _NS
_RPW = _N // _NW
_CH = 16
_NCH = _RPW // _CH
_CSL = _D // _L
_RING = 8
_LEAD = 4


def _body(ids_hbm, tok_hbm, pos_hbm, out_hbm, ids_v, pos_v, *rest):
    bufs, gsems, ssems = rest[:_RING], rest[_RING:2 * _RING], rest[2 * _RING:]
    wid = lax.axis_index("s") * _NC + lax.axis_index("c")
    base = wid * _RPW
    pltpu.sync_copy(pos_hbm, pos_v)
    pltpu.sync_copy(ids_hbm.at[wid], ids_v)

    def gather(k, b):
        idx = ids_v.at[lax.div(k, 8), pl.ds(lax.rem(k, 8) * _CH, _CH)]
        return pltpu.make_async_copy(tok_hbm.at[idx], bufs[b], gsems[b])

    def scatter(k, b):
        return pltpu.make_async_copy(
            bufs[b], out_hbm.at[pl.ds(base + k * _CH, _CH)], ssems[b])

    for b in range(_LEAD):
        gather(b, b).start()

    def step(h, _):
        for b in range(_RING):
            k = h * _RING + b
            gather(k, b).wait()
            s0 = lax.rem(k * _CH, _S)

            @plsc.parallel_loop(0, _CH, carry=s0)
            def _row(i, s):
                buf = bufs[b]
                for j in range(_CSL):
                    sl = pl.ds(j * _L, _L)
                    buf[i, sl] = buf[i, sl] + pos_v[s, sl]
                s = s + 1
                return jnp.where(s == _S, 0, s)

            scatter(k, b).start()

            b2 = (b + _LEAD) % _RING
            @pl.when(k >= _RING - _LEAD)
            def _():
                scatter(k - (_RING - _LEAD), b2).wait()
            @pl.when(k + _LEAD < _NCH)
            def _():
                gather(k + _LEAD, b2).start()
        return 0

    lax.fori_loop(0, _NCH // _RING, step, 0)
    for t in range(_RING - _LEAD, 0, -1):
        b = (_NCH - t) % _RING
        scatter(_NCH - t, b).wait()


def kernel(input_ids, token_table, position_table):
    ids_flat = input_ids.astype(jnp.int32).reshape(_NW, _RPW // 128, 128)
    mesh = plsc.VectorSubcoreMesh(core_axis_name="c", subcore_axis_name="s")
    f = pl.kernel(
        _body,
        out_type=jax.ShapeDtypeStruct((_N, _D), jnp.float32),
        mesh=mesh,
        scratch_types=(
            [pltpu.VMEM((_RPW // 128, 128), jnp.int32),
             pltpu.VMEM((_S, _D), jnp.float32)]
            + [pltpu.VMEM((_CH, _D), jnp.float32) for _ in range(_RING)]
            + [pltpu.SemaphoreType.DMA for _ in range(2 * _RING)]
        ),
    )
    out = f(ids_flat, token_table, position_table)
    return out.reshape(_B, _S, _D)


# speedup vs baseline: 1.2284x; 1.2284x over previous
import jax
import jax.numpy as jnp
from jax import lax
from jax.experimental import pallas as pl
from jax.experimental.pallas import tpu as pltpu
from jax.experimental.pallas import tpu_sc as plsc

_TOKENS = 49408
_D = 512
_S = 77
_B = 4096
_N = _B * _S

_info = plsc.get_sparse_core_info()
_NC, _NS, _L = _info.num_cores, _info.num_subcores, _info.num_lanes
_NW = _NC * _NS
_RPW = _N // _NW
_CH = 32
_NCH = _RPW // _CH
_CSL = _D // _L
_NG = 4
_NO = 2
_PW = 16


def _body(ids_hbm, tok_hbm, pos_hbm, out_hbm, ids_v, pos_v, i0, i1,
          g0, g1, g2, g3, o0, o1, gs0, gs1, gs2, gs3, ss0, ss1):
    gbufs, gsems = (g0, g1, g2, g3), (gs0, gs1, gs2, gs3)
    obufs, ssems, ibufs = (o0, o1), (ss0, ss1), (i0, i1)
    wid = lax.axis_index("s") * _NC + lax.axis_index("c")
    base = wid * _RPW
    a0 = pl.multiple_of((lax.shift_right_logical(base, 12) // 8) * 8, 8)
    pltpu.sync_copy(pos_hbm.at[pl.ds(a0, _PW)], pos_v)
    pltpu.sync_copy(ids_hbm.at[wid], ids_v)

    def gather(k, b):
        idx = ids_v.at[lax.div(k, 4), pl.ds(lax.rem(k, 4) * _CH, _CH)]
        return pltpu.make_async_copy(tok_hbm.at[idx], gbufs[b], gsems[b])

    def scatter(k, o):
        return pltpu.make_async_copy(obufs[o], out_hbm.at[ibufs[o]], ssems[o])

    for b in range(_NG):
        gather(b, b).start()

    iota = lax.iota(jnp.int32, _L)

    def step(h, _):
        for b in range(_NG):
            k = h * _NG + b
            o = b % _NO
            gather(k, b).wait()
            @pl.when(k >= _NO)
            def _():
                scatter(k - _NO, o).wait()

            m0 = base + k * _CH
            s = lax.shift_right_logical(m0, 12)
            ls = s - a0
            rbase = (m0 - s * _B) * _S + s
            for m2 in range(_CH // _L):
                ibufs[o][pl.ds(m2 * _L, _L)] = rbase + (iota + m2 * _L) * _S

            gb, ob = gbufs[b], obufs[o]

            @plsc.parallel_loop(0, _CH)
            def _row(i):
                for j in range(_CSL):
                    sl = pl.ds(j * _L, _L)
                    ob[i, sl] = gb[i, sl] + pos_v[ls, sl]

            scatter(k, o).start()
            @pl.when(k + _NG < _NCH)
            def _():
                gather(k + _NG, b).start()
        return 0

    lax.fori_loop(0, _NCH // _NG, step, 0)
    scatter(_NCH - 2, 0).wait()
    scatter(_NCH - 1, 1).wait()


def kernel(input_ids, token_table, position_table):
    ids_sm = input_ids.astype(jnp.int32).T.reshape(_NW, _RPW // 128, 128)
    pos_p = jnp.pad(position_table, ((0, 88 - _S), (0, 0)))
    mesh = plsc.VectorSubcoreMesh(core_axis_name="c", subcore_axis_name="s")
    f = pl.kernel(
        _body,
        out_type=jax.ShapeDtypeStruct((_N, _D), jnp.float32),
        mesh=mesh,
        scratch_types=(
            [pltpu.VMEM((_RPW // 128, 128), jnp.int32),
             pltpu.VMEM((_PW, _D), jnp.float32)]
            + [pltpu.VMEM((_CH,), jnp.int32) for _ in range(_NO)]
            + [pltpu.VMEM((_CH, _D), jnp.float32) for _ in range(_NG + _NO)]
            + [pltpu.SemaphoreType.DMA for _ in range(_NG + _NO)]
        ),
    )
    out = f(ids_sm, token_table, pos_p)
    return out.reshape(_B, _S, _D)
